# baseline (speedup 1.0000x reference)
import functools

import jax
import jax.numpy as jnp
from jax import lax
from jax.experimental import pallas as pl
from jax.experimental.pallas import tpu as pltpu
from jax.experimental.pallas import tpu_sc as plsc

N_NODES = 10000
N_EDGES = 160000
_NW = 32
_GCHUNK = 40
_SCHUNK = 80
_EBLOCK = 320
_NROWS_CP = 1000


def _leaky(x):
    return jnp.where(x >= 0, x, 0.01 * x)


def _dot(a, b):
    return jnp.dot(a, b, preferred_element_type=jnp.float32)


def _dotb(a, wref):
    return jnp.dot(a.astype(jnp.bfloat16), wref[...],
                   preferred_element_type=jnp.float32)


def _bf(w):
    return w.astype(jnp.bfloat16)


def _pack_cols(t):
    w = t.shape[1]
    pair = jnp.stack([t[:, : w // 2], t[:, w // 2:]], axis=-1)
    return jax.lax.bitcast_convert_type(pair, jnp.int32)


def _unpack_cols(x):
    lo = jax.lax.bitcast_convert_type(jnp.left_shift(x, 16), jnp.float32)
    hi = jax.lax.bitcast_convert_type(
        jnp.bitwise_and(x, jnp.int32(-65536)), jnp.float32)
    return jnp.concatenate([lo, hi], axis=1)


def _mlp_body(n_layers):
    def body(*refs):
        x_ref = refs[0]
        out_ref = refs[-1]
        h = x_ref[...]
        for i in range(n_layers):
            w = refs[1 + 2 * i][...]
            b = refs[2 + 2 * i][...]
            h = _leaky(_dot(h, w) + b)
        out_ref[...] = h
    return body


def _run_mlp(x, Ws, bs, block):
    rows, din = x.shape
    nl = len(Ws)
    dout = Ws[-1].shape[1]
    in_specs = [pl.BlockSpec((block, din), lambda i: (i, 0))]
    args = [x]
    for w, b in zip(Ws, bs):
        in_specs.append(pl.BlockSpec(w.shape, lambda i: (0, 0)))
        args.append(w)
        in_specs.append(pl.BlockSpec((1, b.shape[0]), lambda i: (0, 0)))
        args.append(b.reshape(1, -1))
    return pl.pallas_call(
        _mlp_body(nl),
        grid=(rows // block,),
        in_specs=in_specs,
        out_specs=pl.BlockSpec((block, dout), lambda i: (i, 0)),
        out_shape=jax.ShapeDtypeStruct((rows, dout), jnp.float32),
    )(*args)


def _node_proj(h_n, wa, wcat, block=400):
    n = h_n.shape[0]
    da = wa.shape[1]
    dc = wcat.shape[1]

    def body(h_ref, a_ref, c_ref, u_ref, t_ref):
        h = h_ref[...]
        u_ref[...] = _dot(h, a_ref[...]).astype(jnp.bfloat16)
        t_ref[...] = _dot(h, c_ref[...]).astype(jnp.bfloat16)

    return pl.pallas_call(
        body,
        grid=(n // block,),
        in_specs=[
            pl.BlockSpec((block, h_n.shape[1]), lambda i: (i, 0)),
            pl.BlockSpec(wa.shape, lambda i: (0, 0)),
            pl.BlockSpec(wcat.shape, lambda i: (0, 0)),
        ],
        out_specs=[
            pl.BlockSpec((block, da), lambda i: (i, 0)),
            pl.BlockSpec((block, dc), lambda i: (i, 0)),
        ],
        out_shape=[
            jax.ShapeDtypeStruct((n, da), jnp.bfloat16),
            jax.ShapeDtypeStruct((n, dc), jnp.bfloat16),
        ],
    )(h_n, wa, wcat)


def _node_update(flow, wn, bn, block=1000):
    def body(f_ref, w_ref, b_ref, o_ref):
        o_ref[...] = jnp.maximum(_dot(f_ref[...], w_ref[...]) + b_ref[...], 0.0)

    n, d = flow.shape
    dout = wn.shape[1]
    return pl.pallas_call(
        body,
        grid=(n // block,),
        in_specs=[
            pl.BlockSpec((block, d), lambda i: (i, 0)),
            pl.BlockSpec(wn.shape, lambda i: (0, 0)),
            pl.BlockSpec((1, dout), lambda i: (0, 0)),
        ],
        out_specs=pl.BlockSpec((block, dout), lambda i: (i, 0)),
        out_shape=jax.ShapeDtypeStruct((n, dout), jnp.float32),
    )(flow, wn, bn.reshape(1, -1))


def _edge_step1(gs, gd, ie, src2, dst2, ew, fw):
    nb = N_EDGES // _EBLOCK

    def body(gs_r, gd_r, ie_r, s_r, d_r,
             cc, b1, w2, b2, w3, b3, w4, b4,
             do, bo1, wo2, bo2, di, bi1, wi2, bi2,
             he_out, f_out):
        gdb = _unpack_cols(gd_r[...])
        z = (_unpack_cols(gs_r[...]) + gdb[:, :1024]
             + _dot(ie_r[...], cc[...]) + b1[...])
        a = _leaky(z)
        a = _leaky(_dot(a, w2[...]) + b2[...])
        a = _leaky(_dot(a, w3[...]) + b3[...])
        hen = _leaky(_dot(a, w4[...]) + b4[...])
        he_out[...] = hen
        zo = gdb[:, 1024:2048] + _dot(hen, do[...]) + bo1[...]
        fo = _leaky(_dot(_leaky(zo), wo2[...]) + bo2[...])
        zi = gdb[:, 2048:] + _dot(hen, di[...]) + bi1[...]
        fi = _leaky(_dot(_leaky(zi), wi2[...]) + bi2[...])
        s = s_r[...]
        d = d_r[...]
        fo = jnp.where(s < d, fo, 0.0)
        fi = jnp.where(s > d, fi, 0.0)
        f_out[...] = jnp.concatenate([fi, fo], axis=1)

    wargs = [ew["Ccomb"], ew["b1"], ew["W2"], ew["b2"], ew["W3"], ew["b3"],
             ew["W4"], ew["b4"],
             fw["Do"], fw["bo1"], fw["Wo2"], fw["bo2"],
             fw["Di"], fw["bi1"], fw["Wi2"], fw["bi2"]]
    w_specs = [pl.BlockSpec(w.shape, lambda i: (0,) * w.ndim) for w in wargs]
    return pl.pallas_call(
        body,
        grid=(nb,),
        in_specs=[
            pl.BlockSpec((_EBLOCK, 512), lambda i: (i, 0)),
            pl.BlockSpec((_EBLOCK, 1536), lambda i: (i, 0)),
            pl.BlockSpec((_EBLOCK, 256), lambda i: (i, 0)),
            pl.BlockSpec((_EBLOCK, 1), lambda i: (i, 0)),
            pl.BlockSpec((_EBLOCK, 1), lambda i: (i, 0)),
        ] + w_specs,
        out_specs=[
            pl.BlockSpec((_EBLOCK, 256), lambda i: (i, 0)),
            pl.BlockSpec((_EBLOCK, 1024), lambda i: (i, 0)),
        ],
        out_shape=[
            jax.ShapeDtypeStruct((N_EDGES, 256), jnp.float32),
            jax.ShapeDtypeStruct((N_EDGES, 1024), jnp.float32),
        ],
    )(gs, gd, ie, src2, dst2, *wargs)


def _edge_step2(gs, gdv, ie, he, ew, cw):
    nb = N_EDGES // _EBLOCK

    def body(gs_r, gdv_r, ie_r, he_r,
             c1, c2, b1, w2, b2, w3, b3, w4, b4,
             k1, kb1, k2, kb2, k3, kb3, k4, kb4,
             cls_out):
        z = (_unpack_cols(gs_r[...]) + _unpack_cols(gdv_r[...])
             + _dot(ie_r[...], c1[...]) + _dot(he_r[...], c2[...]) + b1[...])
        a = _leaky(z)
        a = _leaky(_dot(a, w2[...]) + b2[...])
        a = _leaky(_dot(a, w3[...]) + b3[...])
        hen = _leaky(_dot(a, w4[...]) + b4[...])
        c = _leaky(_dot(hen, k1[...]) + kb1[...])
        c = _leaky(_dot(c, k2[...]) + kb2[...])
        c = _leaky(_dot(c, k3[...]) + kb3[...])
        cls_out[...] = _leaky(_dot(c, k4[...]) + kb4[...])

    wargs = [ew["C1"], ew["C2"], ew["b1"], ew["W2"], ew["b2"], ew["W3"],
             ew["b3"], ew["W4"], ew["b4"],
             cw["W"][0], cw["b"][0].reshape(1, -1),
             cw["W"][1], cw["b"][1].reshape(1, -1),
             cw["W"][2], cw["b"][2].reshape(1, -1),
             cw["W"][3], cw["b"][3].reshape(1, -1)]
    w_specs = [pl.BlockSpec(w.shape, lambda i: (0,) * w.ndim) for w in wargs]
    return pl.pallas_call(
        body,
        grid=(nb,),
        in_specs=[
            pl.BlockSpec((_EBLOCK, 512), lambda i: (i, 0)),
            pl.BlockSpec((_EBLOCK, 512), lambda i: (i, 0)),
            pl.BlockSpec((_EBLOCK, 256), lambda i: (i, 0)),
            pl.BlockSpec((_EBLOCK, 256), lambda i: (i, 0)),
        ] + w_specs,
        out_specs=pl.BlockSpec((_EBLOCK, 256), lambda i: (i, 0)),
        out_shape=jax.ShapeDtypeStruct((N_EDGES, 256), jnp.float32),
    )(gs, gdv, ie, he, *wargs)


def _sc_gather(table, idx3, width):
    iters = idx3.shape[1]
    per_w = iters * _GCHUNK
    rows = _NW * per_w
    dt = table.dtype
    dbuf = width <= 512
    mesh = plsc.VectorSubcoreMesh(core_axis_name="c", subcore_axis_name="s")
    scratch = [pltpu.VMEM((iters, _GCHUNK), jnp.int32),
               pltpu.VMEM((_GCHUNK, width), dt),
               pltpu.SemaphoreType.DMA]
    if dbuf:
        scratch += [pltpu.VMEM((_GCHUNK, width), dt), pltpu.SemaphoreType.DMA]

    @functools.partial(
        pl.kernel,
        mesh=mesh,
        out_type=jax.ShapeDtypeStruct((rows, width), dt),
        scratch_types=scratch,
    )
    def gk(table_hbm, idx_hbm, out_hbm, idx_v, rows_a, sem_a, *rest):
        wid = lax.axis_index("s") * 2 + lax.axis_index("c")
        base = wid * per_w
        pltpu.sync_copy(idx_hbm.at[wid], idx_v)

        if dbuf:
            rows_b, sem_b = rest

            def pair(j, carry):
                i0 = j * 2
                e0 = pl.multiple_of(base + i0 * _GCHUNK, 8)
                e1 = pl.multiple_of(base + (i0 + 1) * _GCHUNK, 8)
                ca = pltpu.async_copy(table_hbm.at[idx_v.at[i0]], rows_a, sem_a)
                cb = pltpu.async_copy(table_hbm.at[idx_v.at[i0 + 1]], rows_b,
                                      sem_b)
                ca.wait()
                pltpu.sync_copy(rows_a, out_hbm.at[pl.ds(e0, _GCHUNK)])
                cb.wait()
                pltpu.sync_copy(rows_b, out_hbm.at[pl.ds(e1, _GCHUNK)])
                return carry

            lax.fori_loop(0, iters // 2, pair, 0)
            if iters % 2:
                et = pl.multiple_of(base + (iters - 1) * _GCHUNK, 8)
                pltpu.async_copy(
                    table_hbm.at[idx_v.at[iters - 1]], rows_a, sem_a).wait()
                pltpu.sync_copy(rows_a, out_hbm.at[pl.ds(et, _GCHUNK)])
        else:
            def step(i, carry):
                e0 = pl.multiple_of(base + i * _GCHUNK, 8)
                pltpu.async_copy(table_hbm.at[idx_v.at[i]], rows_a, sem_a).wait()
                pltpu.sync_copy(rows_a, out_hbm.at[pl.ds(e0, _GCHUNK)])
                return carry

            lax.fori_loop(0, iters, step, 0)

    return gk(table, idx3)


def _sc_scatter_sum(f, idx3, zeros_tile):
    iters = idx3.shape[1]
    per_tile = iters * _SCHUNK
    mesh = plsc.VectorSubcoreMesh(core_axis_name="c", subcore_axis_name="s")

    @functools.partial(
        pl.kernel,
        mesh=mesh,
        out_type=jax.ShapeDtypeStruct((N_NODES, 1024), jnp.float32),
        scratch_types=[
            pltpu.VMEM((iters, _SCHUNK), jnp.int32),
            pltpu.VMEM((_SCHUNK, 128), jnp.float32),
            pltpu.VMEM((_SCHUNK, 128), jnp.float32),
            pltpu.VMEM_SHARED((N_NODES, 128), jnp.float32),
            pltpu.SemaphoreType.DMA,
            pltpu.SemaphoreType.DMA,
        ],
    )
    def sk(f_hbm, idx_hbm, z_hbm, out_hbm, idx_v, val_a, val_b, shared,
           sem_a, sem_b):
        c = lax.axis_index("c")
        s = lax.axis_index("s")
        r0 = pl.multiple_of(s * _NROWS_CP, 8)
        pltpu.sync_copy(idx_hbm.at[s], idx_v)
        for cc in range(4):
            col0 = pl.multiple_of((c * 4 + cc) * 128, 128)

            @pl.when(s < 10)
            def _zero():
                pltpu.sync_copy(z_hbm, shared.at[pl.ds(r0, _NROWS_CP)])

            plsc.subcore_barrier()

            def pair(j, carry):
                i0 = j * 2
                e0 = pl.multiple_of(s * per_tile + i0 * _SCHUNK, 8)
                e1 = pl.multiple_of(s * per_tile + (i0 + 1) * _SCHUNK, 8)
                ca = pltpu.async_copy(
                    f_hbm.at[pl.ds(e0, _SCHUNK), pl.ds(col0, 128)], val_a,
                    sem_a)
                cb = pltpu.async_copy(
                    f_hbm.at[pl.ds(e1, _SCHUNK), pl.ds(col0, 128)], val_b,
                    sem_b)
                ca.wait()
                pltpu.sync_copy(val_a, shared.at[idx_v.at[i0]], add=True)
                cb.wait()
                pltpu.sync_copy(val_b, shared.at[idx_v.at[i0 + 1]], add=True)
                return carry

            lax.fori_loop(0, iters // 2, pair, 0)
            if iters % 2:
                et = pl.multiple_of(s * per_tile + (iters - 1) * _SCHUNK, 8)
                pltpu.async_copy(
                    f_hbm.at[pl.ds(et, _SCHUNK), pl.ds(col0, 128)], val_a,
                    sem_a).wait()
                pltpu.sync_copy(val_a, shared.at[idx_v.at[iters - 1]],
                                add=True)
            plsc.subcore_barrier()

            @pl.when(s < 10)
            def _copy_out():
                pltpu.sync_copy(
                    shared.at[pl.ds(r0, _NROWS_CP)],
                    out_hbm.at[pl.ds(r0, _NROWS_CP), pl.ds(col0, 128)])

            plsc.subcore_barrier()

    return sk(f, idx3, zeros_tile)


def kernel(x, edge_index, edge_attr, params):
    p = params
    src = edge_index[0]
    dst = edge_index[1]

    h_e0 = _run_mlp(edge_attr, p["enc_edge"]["W"], p["enc_edge"]["b"], 640)
    h_n = _run_mlp(x, p["enc_node"]["W"], p["enc_node"]["b"], 1000)
    init_e = h_e0

    w1 = p["edge_model"]["W"][0]
    wa, wb = w1[0:1024], w1[1024:2048]
    c1, c2 = w1[2048:2304], w1[2304:2560]
    ew = {
        "C1": c1, "C2": c2, "Ccomb": c1 + c2,
        "b1": p["edge_model"]["b"][0].reshape(1, -1),
        "W2": p["edge_model"]["W"][1], "b2": p["edge_model"]["b"][1].reshape(1, -1),
        "W3": p["edge_model"]["W"][2], "b3": p["edge_model"]["b"][2].reshape(1, -1),
        "W4": p["edge_model"]["W"][3], "b4": p["edge_model"]["b"][3].reshape(1, -1),
    }
    wfo1, wfi1 = p["flow_out"]["W"][0], p["flow_in"]["W"][0]
    fw = {
        "Do": wfo1[1024:], "bo1": p["flow_out"]["b"][0].reshape(1, -1),
        "Wo2": p["flow_out"]["W"][1], "bo2": p["flow_out"]["b"][1].reshape(1, -1),
        "Di": wfi1[1024:], "bi1": p["flow_in"]["b"][0].reshape(1, -1),
        "Wi2": p["flow_in"]["W"][1], "bi2": p["flow_in"]["b"][1].reshape(1, -1),
    }
    wcat1 = jnp.concatenate([wb, wfo1[:1024], wfi1[:1024]], axis=1)

    src2 = src.reshape(N_EDGES, 1)
    dst2 = dst.reshape(N_EDGES, 1)
    src_g = src.reshape(_NW, N_EDGES // _NW // _GCHUNK, _GCHUNK)
    dst_g = dst.reshape(_NW, N_EDGES // _NW // _GCHUNK, _GCHUNK)
    src_s = src.reshape(16, N_EDGES // 16 // _SCHUNK, _SCHUNK)
    zeros_tile = jnp.zeros((_NROWS_CP, 128), jnp.float32)

    u, td = _node_proj(h_n, wa, wcat1)
    gs = _sc_gather(_pack_cols(u), src_g, 512)
    gd = _sc_gather(_pack_cols(td), dst_g, 1536)
    h_e, f = _edge_step1(gs, gd, init_e, src2, dst2, ew, fw)
    flow = _sc_scatter_sum(f, src_s, zeros_tile)
    h_n = _node_update(flow, p["node_mlp"]["W"][0], p["node_mlp"]["b"][0])

    u2, v2 = _node_proj(h_n, wa, wb)
    gs2 = _sc_gather(_pack_cols(u2), src_g, 512)
    gd2 = _sc_gather(_pack_cols(v2), dst_g, 512)
    out = _edge_step2(gs2, gd2, init_e, h_e, ew, p["classifier"])
    return out


# speedup vs baseline: 1.0871x; 1.0871x over previous
import functools

import jax
import jax.numpy as jnp
from jax import lax
from jax.experimental import pallas as pl
from jax.experimental.pallas import tpu as pltpu
from jax.experimental.pallas import tpu_sc as plsc

N_NODES = 10000
N_EDGES = 160000
_NW = 32
_GCHUNK = 40
_SCHUNK = 80
_EBLOCK = 640
_NROWS_CP = 1000


def _leaky(x):
    return jnp.where(x >= 0, x, 0.01 * x)


def _dot(a, b):
    return jnp.dot(a, b, preferred_element_type=jnp.float32)


def _dotb(a, wref):
    return jnp.dot(a.astype(jnp.bfloat16), wref[...],
                   preferred_element_type=jnp.float32)


def _bf(w):
    return w.astype(jnp.bfloat16)


def _pack_cols(t):
    w = t.shape[1]
    pair = jnp.stack([t[:, : w // 2], t[:, w // 2:]], axis=-1)
    return jax.lax.bitcast_convert_type(pair, jnp.int32)


def _unpack_cols(x):
    lo = jax.lax.bitcast_convert_type(jnp.left_shift(x, 16), jnp.float32)
    hi = jax.lax.bitcast_convert_type(
        jnp.bitwise_and(x, jnp.int32(-65536)), jnp.float32)
    return jnp.concatenate([lo, hi], axis=1)


def _mlp_body(n_layers):
    def body(*refs):
        x_ref = refs[0]
        out_ref = refs[-1]
        h = x_ref[...]
        for i in range(n_layers):
            w = refs[1 + 2 * i][...]
            b = refs[2 + 2 * i][...]
            h = _leaky(_dot(h, w) + b)
        out_ref[...] = h
    return body


def _run_mlp(x, Ws, bs, block):
    rows, din = x.shape
    nl = len(Ws)
    dout = Ws[-1].shape[1]
    in_specs = [pl.BlockSpec((block, din), lambda i: (i, 0))]
    args = [x]
    for w, b in zip(Ws, bs):
        in_specs.append(pl.BlockSpec(w.shape, lambda i: (0, 0)))
        args.append(w)
        in_specs.append(pl.BlockSpec((1, b.shape[0]), lambda i: (0, 0)))
        args.append(b.reshape(1, -1))
    return pl.pallas_call(
        _mlp_body(nl),
        grid=(rows // block,),
        in_specs=in_specs,
        out_specs=pl.BlockSpec((block, dout), lambda i: (i, 0)),
        out_shape=jax.ShapeDtypeStruct((rows, dout), jnp.float32),
    )(*args)


def _node_proj(h_n, wa, wcat, block=400):
    n = h_n.shape[0]
    da = wa.shape[1]
    dc = wcat.shape[1]

    def body(h_ref, a_ref, c_ref, u_ref, t_ref):
        h = h_ref[...]
        u_ref[...] = _dot(h, a_ref[...]).astype(jnp.bfloat16)
        t_ref[...] = _dot(h, c_ref[...]).astype(jnp.bfloat16)

    return pl.pallas_call(
        body,
        grid=(n // block,),
        in_specs=[
            pl.BlockSpec((block, h_n.shape[1]), lambda i: (i, 0)),
            pl.BlockSpec(wa.shape, lambda i: (0, 0)),
            pl.BlockSpec(wcat.shape, lambda i: (0, 0)),
        ],
        out_specs=[
            pl.BlockSpec((block, da), lambda i: (i, 0)),
            pl.BlockSpec((block, dc), lambda i: (i, 0)),
        ],
        out_shape=[
            jax.ShapeDtypeStruct((n, da), jnp.bfloat16),
            jax.ShapeDtypeStruct((n, dc), jnp.bfloat16),
        ],
    )(h_n, wa, wcat)


def _node_update(flow, wn, bn, block=1000):
    def body(f_ref, w_ref, b_ref, o_ref):
        o_ref[...] = jnp.maximum(_dot(f_ref[...], w_ref[...]) + b_ref[...], 0.0)

    n, d = flow.shape
    dout = wn.shape[1]
    return pl.pallas_call(
        body,
        grid=(n // block,),
        in_specs=[
            pl.BlockSpec((block, d), lambda i: (i, 0)),
            pl.BlockSpec(wn.shape, lambda i: (0, 0)),
            pl.BlockSpec((1, dout), lambda i: (0, 0)),
        ],
        out_specs=pl.BlockSpec((block, dout), lambda i: (i, 0)),
        out_shape=jax.ShapeDtypeStruct((n, dout), jnp.float32),
    )(flow, wn, bn.reshape(1, -1))


def _edge_step1(gs, gd, ie, src2, dst2, ew, fw):
    nb = N_EDGES // _EBLOCK

    def body(gs_r, gd_r, ie_r, s_r, d_r,
             cc, b1, w2, b2, w3, b3, w4, b4,
             do, bo1, wo2, bo2, di, bi1, wi2, bi2,
             he_out, f_out):
        gdb = _unpack_cols(gd_r[...])
        z = (_unpack_cols(gs_r[...]) + gdb[:, :1024]
             + _dot(ie_r[...], cc[...]) + b1[...])
        a = _leaky(z)
        a = _leaky(_dot(a, w2[...]) + b2[...])
        a = _leaky(_dot(a, w3[...]) + b3[...])
        hen = _leaky(_dot(a, w4[...]) + b4[...])
        he_out[...] = hen
        zo = gdb[:, 1024:2048] + _dot(hen, do[...]) + bo1[...]
        fo = _leaky(_dot(_leaky(zo), wo2[...]) + bo2[...])
        zi = gdb[:, 2048:] + _dot(hen, di[...]) + bi1[...]
        fi = _leaky(_dot(_leaky(zi), wi2[...]) + bi2[...])
        s = s_r[...]
        d = d_r[...]
        fo = jnp.where(s < d, fo, 0.0)
        fi = jnp.where(s > d, fi, 0.0)
        f_out[...] = jnp.concatenate([fi, fo], axis=1)

    wargs = [ew["Ccomb"], ew["b1"], ew["W2"], ew["b2"], ew["W3"], ew["b3"],
             ew["W4"], ew["b4"],
             fw["Do"], fw["bo1"], fw["Wo2"], fw["bo2"],
             fw["Di"], fw["bi1"], fw["Wi2"], fw["bi2"]]
    w_specs = [pl.BlockSpec(w.shape, lambda i: (0,) * w.ndim) for w in wargs]
    return pl.pallas_call(
        body,
        grid=(nb,),
        in_specs=[
            pl.BlockSpec((_EBLOCK, 512), lambda i: (i, 0)),
            pl.BlockSpec((_EBLOCK, 1536), lambda i: (i, 0)),
            pl.BlockSpec((_EBLOCK, 256), lambda i: (i, 0)),
            pl.BlockSpec((_EBLOCK, 1), lambda i: (i, 0)),
            pl.BlockSpec((_EBLOCK, 1), lambda i: (i, 0)),
        ] + w_specs,
        out_specs=[
            pl.BlockSpec((_EBLOCK, 256), lambda i: (i, 0)),
            pl.BlockSpec((_EBLOCK, 1024), lambda i: (i, 0)),
        ],
        out_shape=[
            jax.ShapeDtypeStruct((N_EDGES, 256), jnp.float32),
            jax.ShapeDtypeStruct((N_EDGES, 1024), jnp.float32),
        ],
    )(gs, gd, ie, src2, dst2, *wargs)


def _edge_step2(gs, gdv, ie, he, ew, cw):
    nb = N_EDGES // _EBLOCK

    def body(gs_r, gdv_r, ie_r, he_r,
             c1, c2, b1, w2, b2, w3, b3, w4, b4,
             k1, kb1, k2, kb2, k3, kb3, k4, kb4,
             cls_out):
        z = (_unpack_cols(gs_r[...]) + _unpack_cols(gdv_r[...])
             + _dot(ie_r[...], c1[...]) + _dot(he_r[...], c2[...]) + b1[...])
        a = _leaky(z)
        a = _leaky(_dot(a, w2[...]) + b2[...])
        a = _leaky(_dot(a, w3[...]) + b3[...])
        hen = _leaky(_dot(a, w4[...]) + b4[...])
        c = _leaky(_dot(hen, k1[...]) + kb1[...])
        c = _leaky(_dot(c, k2[...]) + kb2[...])
        c = _leaky(_dot(c, k3[...]) + kb3[...])
        cls_out[...] = _leaky(_dot(c, k4[...]) + kb4[...])

    wargs = [ew["C1"], ew["C2"], ew["b1"], ew["W2"], ew["b2"], ew["W3"],
             ew["b3"], ew["W4"], ew["b4"],
             cw["W"][0], cw["b"][0].reshape(1, -1),
             cw["W"][1], cw["b"][1].reshape(1, -1),
             cw["W"][2], cw["b"][2].reshape(1, -1),
             cw["W"][3], cw["b"][3].reshape(1, -1)]
    w_specs = [pl.BlockSpec(w.shape, lambda i: (0,) * w.ndim) for w in wargs]
    return pl.pallas_call(
        body,
        grid=(nb,),
        in_specs=[
            pl.BlockSpec((_EBLOCK, 512), lambda i: (i, 0)),
            pl.BlockSpec((_EBLOCK, 512), lambda i: (i, 0)),
            pl.BlockSpec((_EBLOCK, 256), lambda i: (i, 0)),
            pl.BlockSpec((_EBLOCK, 256), lambda i: (i, 0)),
        ] + w_specs,
        out_specs=pl.BlockSpec((_EBLOCK, 256), lambda i: (i, 0)),
        out_shape=jax.ShapeDtypeStruct((N_EDGES, 256), jnp.float32),
    )(gs, gdv, ie, he, *wargs)


def _sc_gather(table, idx3, width):
    iters = idx3.shape[1]
    per_w = iters * _GCHUNK
    rows = _NW * per_w
    dt = table.dtype
    dbuf = width <= 512
    mesh = plsc.VectorSubcoreMesh(core_axis_name="c", subcore_axis_name="s")
    scratch = [pltpu.VMEM((iters, _GCHUNK), jnp.int32),
               pltpu.VMEM((_GCHUNK, width), dt),
               pltpu.SemaphoreType.DMA]
    if dbuf:
        scratch += [pltpu.VMEM((_GCHUNK, width), dt), pltpu.SemaphoreType.DMA]

    @functools.partial(
        pl.kernel,
        mesh=mesh,
        out_type=jax.ShapeDtypeStruct((rows, width), dt),
        scratch_types=scratch,
    )
    def gk(table_hbm, idx_hbm, out_hbm, idx_v, rows_a, sem_a, *rest):
        wid = lax.axis_index("s") * 2 + lax.axis_index("c")
        base = wid * per_w
        pltpu.sync_copy(idx_hbm.at[wid], idx_v)

        if dbuf:
            rows_b, sem_b = rest

            def pair(j, carry):
                i0 = j * 2
                e0 = pl.multiple_of(base + i0 * _GCHUNK, 8)
                e1 = pl.multiple_of(base + (i0 + 1) * _GCHUNK, 8)
                ca = pltpu.async_copy(table_hbm.at[idx_v.at[i0]], rows_a, sem_a)
                cb = pltpu.async_copy(table_hbm.at[idx_v.at[i0 + 1]], rows_b,
                                      sem_b)
                ca.wait()
                pltpu.sync_copy(rows_a, out_hbm.at[pl.ds(e0, _GCHUNK)])
                cb.wait()
                pltpu.sync_copy(rows_b, out_hbm.at[pl.ds(e1, _GCHUNK)])
                return carry

            lax.fori_loop(0, iters // 2, pair, 0)
            if iters % 2:
                et = pl.multiple_of(base + (iters - 1) * _GCHUNK, 8)
                pltpu.async_copy(
                    table_hbm.at[idx_v.at[iters - 1]], rows_a, sem_a).wait()
                pltpu.sync_copy(rows_a, out_hbm.at[pl.ds(et, _GCHUNK)])
        else:
            def step(i, carry):
                e0 = pl.multiple_of(base + i * _GCHUNK, 8)
                pltpu.async_copy(table_hbm.at[idx_v.at[i]], rows_a, sem_a).wait()
                pltpu.sync_copy(rows_a, out_hbm.at[pl.ds(e0, _GCHUNK)])
                return carry

            lax.fori_loop(0, iters, step, 0)

    return gk(table, idx3)


def _sc_scatter_sum(f, idx3, zeros_tile):
    iters = idx3.shape[1]
    per_tile = iters * _SCHUNK
    mesh = plsc.VectorSubcoreMesh(core_axis_name="c", subcore_axis_name="s")

    @functools.partial(
        pl.kernel,
        mesh=mesh,
        out_type=jax.ShapeDtypeStruct((N_NODES, 1024), jnp.float32),
        scratch_types=[
            pltpu.VMEM((iters, _SCHUNK), jnp.int32),
            pltpu.VMEM((_SCHUNK, 128), jnp.float32),
            pltpu.VMEM((_SCHUNK, 128), jnp.float32),
            pltpu.VMEM_SHARED((N_NODES, 128), jnp.float32),
            pltpu.SemaphoreType.DMA,
            pltpu.SemaphoreType.DMA,
        ],
    )
    def sk(f_hbm, idx_hbm, z_hbm, out_hbm, idx_v, val_a, val_b, shared,
           sem_a, sem_b):
        c = lax.axis_index("c")
        s = lax.axis_index("s")
        r0 = pl.multiple_of(s * _NROWS_CP, 8)
        pltpu.sync_copy(idx_hbm.at[s], idx_v)
        for cc in range(4):
            col0 = pl.multiple_of((c * 4 + cc) * 128, 128)

            @pl.when(s < 10)
            def _zero():
                pltpu.sync_copy(z_hbm, shared.at[pl.ds(r0, _NROWS_CP)])

            plsc.subcore_barrier()

            def pair(j, carry):
                i0 = j * 2
                e0 = pl.multiple_of(s * per_tile + i0 * _SCHUNK, 8)
                e1 = pl.multiple_of(s * per_tile + (i0 + 1) * _SCHUNK, 8)
                ca = pltpu.async_copy(
                    f_hbm.at[pl.ds(e0, _SCHUNK), pl.ds(col0, 128)], val_a,
                    sem_a)
                cb = pltpu.async_copy(
                    f_hbm.at[pl.ds(e1, _SCHUNK), pl.ds(col0, 128)], val_b,
                    sem_b)
                ca.wait()
                pltpu.sync_copy(val_a, shared.at[idx_v.at[i0]], add=True)
                cb.wait()
                pltpu.sync_copy(val_b, shared.at[idx_v.at[i0 + 1]], add=True)
                return carry

            lax.fori_loop(0, iters // 2, pair, 0)
            if iters % 2:
                et = pl.multiple_of(s * per_tile + (iters - 1) * _SCHUNK, 8)
                pltpu.async_copy(
                    f_hbm.at[pl.ds(et, _SCHUNK), pl.ds(col0, 128)], val_a,
                    sem_a).wait()
                pltpu.sync_copy(val_a, shared.at[idx_v.at[iters - 1]],
                                add=True)
            plsc.subcore_barrier()

            @pl.when(s < 10)
            def _copy_out():
                pltpu.sync_copy(
                    shared.at[pl.ds(r0, _NROWS_CP)],
                    out_hbm.at[pl.ds(r0, _NROWS_CP), pl.ds(col0, 128)])

            plsc.subcore_barrier()

    return sk(f, idx3, zeros_tile)


def kernel(x, edge_index, edge_attr, params):
    p = params
    src = edge_index[0]
    dst = edge_index[1]

    h_e0 = _run_mlp(edge_attr, p["enc_edge"]["W"], p["enc_edge"]["b"], 640)
    h_n = _run_mlp(x, p["enc_node"]["W"], p["enc_node"]["b"], 1000)
    init_e = h_e0

    w1 = p["edge_model"]["W"][0]
    wa, wb = w1[0:1024], w1[1024:2048]
    c1, c2 = w1[2048:2304], w1[2304:2560]
    ew = {
        "C1": c1, "C2": c2, "Ccomb": c1 + c2,
        "b1": p["edge_model"]["b"][0].reshape(1, -1),
        "W2": p["edge_model"]["W"][1], "b2": p["edge_model"]["b"][1].reshape(1, -1),
        "W3": p["edge_model"]["W"][2], "b3": p["edge_model"]["b"][2].reshape(1, -1),
        "W4": p["edge_model"]["W"][3], "b4": p["edge_model"]["b"][3].reshape(1, -1),
    }
    wfo1, wfi1 = p["flow_out"]["W"][0], p["flow_in"]["W"][0]
    fw = {
        "Do": wfo1[1024:], "bo1": p["flow_out"]["b"][0].reshape(1, -1),
        "Wo2": p["flow_out"]["W"][1], "bo2": p["flow_out"]["b"][1].reshape(1, -1),
        "Di": wfi1[1024:], "bi1": p["flow_in"]["b"][0].reshape(1, -1),
        "Wi2": p["flow_in"]["W"][1], "bi2": p["flow_in"]["b"][1].reshape(1, -1),
    }
    wcat1 = jnp.concatenate([wb, wfo1[:1024], wfi1[:1024]], axis=1)

    src2 = src.reshape(N_EDGES, 1)
    dst2 = dst.reshape(N_EDGES, 1)
    src_g = src.reshape(_NW, N_EDGES // _NW // _GCHUNK, _GCHUNK)
    dst_g = dst.reshape(_NW, N_EDGES // _NW // _GCHUNK, _GCHUNK)
    src_s = src.reshape(16, N_EDGES // 16 // _SCHUNK, _SCHUNK)
    zeros_tile = jnp.zeros((_NROWS_CP, 128), jnp.float32)

    u, td = _node_proj(h_n, wa, wcat1)
    gs = _sc_gather(_pack_cols(u), src_g, 512)
    gd = _sc_gather(_pack_cols(td), dst_g, 1536)
    h_e, f = _edge_step1(gs, gd, init_e, src2, dst2, ew, fw)
    flow = _sc_scatter_sum(f, src_s, zeros_tile)
    h_n = _node_update(flow, p["node_mlp"]["W"][0], p["node_mlp"]["b"][0])

    u2, v2 = _node_proj(h_n, wa, wb)
    gs2 = _sc_gather(_pack_cols(u2), src_g, 512)
    gd2 = _sc_gather(_pack_cols(v2), dst_g, 512)
    out = _edge_step2(gs2, gd2, init_e, h_e, ew, p["classifier"])
    return out


# speedup vs baseline: 1.1839x; 1.0891x over previous
import functools

import jax
import jax.numpy as jnp
from jax import lax
from jax.experimental import pallas as pl
from jax.experimental.pallas import tpu as pltpu
from jax.experimental.pallas import tpu_sc as plsc

N_NODES = 10000
N_EDGES = 160000
_NW = 32
_GCHUNK = 40
_SCHUNK = 80
_EBLOCK = 800
_NROWS_CP = 1000


def _leaky(x):
    return jnp.where(x >= 0, x, 0.01 * x)


def _dot(a, b):
    return jnp.dot(a, b, preferred_element_type=jnp.float32)


def _dotb(a, wref):
    return jnp.dot(a.astype(jnp.bfloat16), wref[...],
                   preferred_element_type=jnp.float32)


def _bf(w):
    return w.astype(jnp.bfloat16)


def _pack_cols(t):
    w = t.shape[1]
    pair = jnp.stack([t[:, : w // 2], t[:, w // 2:]], axis=-1)
    return jax.lax.bitcast_convert_type(pair, jnp.int32)


def _unpack_cols(x):
    lo = jax.lax.bitcast_convert_type(jnp.left_shift(x, 16), jnp.float32)
    hi = jax.lax.bitcast_convert_type(
        jnp.bitwise_and(x, jnp.int32(-65536)), jnp.float32)
    return jnp.concatenate([lo, hi], axis=1)


def _mlp_body(n_layers):
    def body(*refs):
        x_ref = refs[0]
        out_ref = refs[-1]
        h = x_ref[...]
        for i in range(n_layers):
            w = refs[1 + 2 * i][...]
            b = refs[2 + 2 * i][...]
            h = _leaky(_dot(h, w) + b)
        out_ref[...] = h
    return body


def _run_mlp(x, Ws, bs, block):
    rows, din = x.shape
    nl = len(Ws)
    dout = Ws[-1].shape[1]
    in_specs = [pl.BlockSpec((block, din), lambda i: (i, 0))]
    args = [x]
    for w, b in zip(Ws, bs):
        in_specs.append(pl.BlockSpec(w.shape, lambda i: (0, 0)))
        args.append(w)
        in_specs.append(pl.BlockSpec((1, b.shape[0]), lambda i: (0, 0)))
        args.append(b.reshape(1, -1))
    return pl.pallas_call(
        _mlp_body(nl),
        grid=(rows // block,),
        in_specs=in_specs,
        out_specs=pl.BlockSpec((block, dout), lambda i: (i, 0)),
        out_shape=jax.ShapeDtypeStruct((rows, dout), jnp.float32),
    )(*args)


def _node_proj(h_n, wa, wcat, block=400):
    n = h_n.shape[0]
    da = wa.shape[1]
    dc = wcat.shape[1]

    def body(h_ref, a_ref, c_ref, u_ref, t_ref):
        h = h_ref[...]
        u_ref[...] = _dot(h, a_ref[...]).astype(jnp.bfloat16)
        t_ref[...] = _dot(h, c_ref[...]).astype(jnp.bfloat16)

    return pl.pallas_call(
        body,
        grid=(n // block,),
        in_specs=[
            pl.BlockSpec((block, h_n.shape[1]), lambda i: (i, 0)),
            pl.BlockSpec(wa.shape, lambda i: (0, 0)),
            pl.BlockSpec(wcat.shape, lambda i: (0, 0)),
        ],
        out_specs=[
            pl.BlockSpec((block, da), lambda i: (i, 0)),
            pl.BlockSpec((block, dc), lambda i: (i, 0)),
        ],
        out_shape=[
            jax.ShapeDtypeStruct((n, da), jnp.bfloat16),
            jax.ShapeDtypeStruct((n, dc), jnp.bfloat16),
        ],
    )(h_n, wa, wcat)


def _node_update(flow, wn, bn, block=1000):
    def body(f_ref, w_ref, b_ref, o_ref):
        o_ref[...] = jnp.maximum(_dot(f_ref[...], w_ref[...]) + b_ref[...], 0.0)

    n, d = flow.shape
    dout = wn.shape[1]
    return pl.pallas_call(
        body,
        grid=(n // block,),
        in_specs=[
            pl.BlockSpec((block, d), lambda i: (i, 0)),
            pl.BlockSpec(wn.shape, lambda i: (0, 0)),
            pl.BlockSpec((1, dout), lambda i: (0, 0)),
        ],
        out_specs=pl.BlockSpec((block, dout), lambda i: (i, 0)),
        out_shape=jax.ShapeDtypeStruct((n, dout), jnp.float32),
    )(flow, wn, bn.reshape(1, -1))


def _edge_step1(gs, gd, ie, src2, dst2, ew, fw):
    nb = N_EDGES // _EBLOCK

    def body(gs_r, gd_r, ie_r, s_r, d_r,
             cc, b1, w2, b2, w3, b3, w4, b4,
             do, bo1, wo2, bo2, di, bi1, wi2, bi2,
             he_out, f_out):
        gdb = _unpack_cols(gd_r[...])
        z = (_unpack_cols(gs_r[...]) + gdb[:, :1024]
             + _dot(ie_r[...], cc[...]) + b1[...])
        a = _leaky(z)
        a = _leaky(_dot(a, w2[...]) + b2[...])
        a = _leaky(_dot(a, w3[...]) + b3[...])
        hen = _leaky(_dot(a, w4[...]) + b4[...])
        he_out[...] = hen
        zo = gdb[:, 1024:2048] + _dot(hen, do[...]) + bo1[...]
        fo = _leaky(_dot(_leaky(zo), wo2[...]) + bo2[...])
        zi = gdb[:, 2048:] + _dot(hen, di[...]) + bi1[...]
        fi = _leaky(_dot(_leaky(zi), wi2[...]) + bi2[...])
        s = s_r[...]
        d = d_r[...]
        fo = jnp.where(s < d, fo, 0.0)
        fi = jnp.where(s > d, fi, 0.0)
        f_out[...] = jnp.concatenate([fi, fo], axis=1)

    wargs = [ew["Ccomb"], ew["b1"], ew["W2"], ew["b2"], ew["W3"], ew["b3"],
             ew["W4"], ew["b4"],
             fw["Do"], fw["bo1"], fw["Wo2"], fw["bo2"],
             fw["Di"], fw["bi1"], fw["Wi2"], fw["bi2"]]
    w_specs = [pl.BlockSpec(w.shape, lambda i: (0,) * w.ndim) for w in wargs]
    return pl.pallas_call(
        body,
        grid=(nb,),
        in_specs=[
            pl.BlockSpec((_EBLOCK, 512), lambda i: (i, 0)),
            pl.BlockSpec((_EBLOCK, 1536), lambda i: (i, 0)),
            pl.BlockSpec((_EBLOCK, 256), lambda i: (i, 0)),
            pl.BlockSpec((_EBLOCK, 1), lambda i: (i, 0)),
            pl.BlockSpec((_EBLOCK, 1), lambda i: (i, 0)),
        ] + w_specs,
        out_specs=[
            pl.BlockSpec((_EBLOCK, 256), lambda i: (i, 0)),
            pl.BlockSpec((_EBLOCK, 1024), lambda i: (i, 0)),
        ],
        out_shape=[
            jax.ShapeDtypeStruct((N_EDGES, 256), jnp.float32),
            jax.ShapeDtypeStruct((N_EDGES, 1024), jnp.float32),
        ],
    )(gs, gd, ie, src2, dst2, *wargs)


def _edge_step2(gs, gdv, ie, he, ew, cw):
    nb = N_EDGES // _EBLOCK

    def body(gs_r, gdv_r, ie_r, he_r,
             c1, c2, b1, w2, b2, w3, b3, w4, b4,
             k1, kb1, k2, kb2, k3, kb3, k4, kb4,
             cls_out):
        z = (_unpack_cols(gs_r[...]) + _unpack_cols(gdv_r[...])
             + _dot(ie_r[...], c1[...]) + _dot(he_r[...], c2[...]) + b1[...])
        a = _leaky(z)
        a = _leaky(_dot(a, w2[...]) + b2[...])
        a = _leaky(_dot(a, w3[...]) + b3[...])
        hen = _leaky(_dot(a, w4[...]) + b4[...])
        c = _leaky(_dot(hen, k1[...]) + kb1[...])
        c = _leaky(_dot(c, k2[...]) + kb2[...])
        c = _leaky(_dot(c, k3[...]) + kb3[...])
        cls_out[...] = _leaky(_dot(c, k4[...]) + kb4[...])

    wargs = [ew["C1"], ew["C2"], ew["b1"], ew["W2"], ew["b2"], ew["W3"],
             ew["b3"], ew["W4"], ew["b4"],
             cw["W"][0], cw["b"][0].reshape(1, -1),
             cw["W"][1], cw["b"][1].reshape(1, -1),
             cw["W"][2], cw["b"][2].reshape(1, -1),
             cw["W"][3], cw["b"][3].reshape(1, -1)]
    w_specs = [pl.BlockSpec(w.shape, lambda i: (0,) * w.ndim) for w in wargs]
    return pl.pallas_call(
        body,
        grid=(nb,),
        in_specs=[
            pl.BlockSpec((_EBLOCK, 512), lambda i: (i, 0)),
            pl.BlockSpec((_EBLOCK, 512), lambda i: (i, 0)),
            pl.BlockSpec((_EBLOCK, 256), lambda i: (i, 0)),
            pl.BlockSpec((_EBLOCK, 256), lambda i: (i, 0)),
        ] + w_specs,
        out_specs=pl.BlockSpec((_EBLOCK, 256), lambda i: (i, 0)),
        out_shape=jax.ShapeDtypeStruct((N_EDGES, 256), jnp.float32),
    )(gs, gdv, ie, he, *wargs)


def _sc_gather(table, idx3, width):
    iters = idx3.shape[1]
    per_w = iters * _GCHUNK
    rows = _NW * per_w
    dt = table.dtype
    dbuf = width <= 512
    mesh = plsc.VectorSubcoreMesh(core_axis_name="c", subcore_axis_name="s")
    scratch = [pltpu.VMEM((iters, _GCHUNK), jnp.int32),
               pltpu.VMEM((_GCHUNK, width), dt),
               pltpu.SemaphoreType.DMA]
    if dbuf:
        scratch += [pltpu.VMEM((_GCHUNK, width), dt), pltpu.SemaphoreType.DMA]

    @functools.partial(
        pl.kernel,
        mesh=mesh,
        out_type=jax.ShapeDtypeStruct((rows, width), dt),
        scratch_types=scratch,
    )
    def gk(table_hbm, idx_hbm, out_hbm, idx_v, rows_a, sem_a, *rest):
        wid = lax.axis_index("s") * 2 + lax.axis_index("c")
        base = wid * per_w
        pltpu.sync_copy(idx_hbm.at[wid], idx_v)

        if dbuf:
            rows_b, sem_b = rest

            def pair(j, carry):
                i0 = j * 2
                e0 = pl.multiple_of(base + i0 * _GCHUNK, 8)
                e1 = pl.multiple_of(base + (i0 + 1) * _GCHUNK, 8)
                ca = pltpu.async_copy(table_hbm.at[idx_v.at[i0]], rows_a, sem_a)
                cb = pltpu.async_copy(table_hbm.at[idx_v.at[i0 + 1]], rows_b,
                                      sem_b)
                ca.wait()
                pltpu.sync_copy(rows_a, out_hbm.at[pl.ds(e0, _GCHUNK)])
                cb.wait()
                pltpu.sync_copy(rows_b, out_hbm.at[pl.ds(e1, _GCHUNK)])
                return carry

            lax.fori_loop(0, iters // 2, pair, 0)
            if iters % 2:
                et = pl.multiple_of(base + (iters - 1) * _GCHUNK, 8)
                pltpu.async_copy(
                    table_hbm.at[idx_v.at[iters - 1]], rows_a, sem_a).wait()
                pltpu.sync_copy(rows_a, out_hbm.at[pl.ds(et, _GCHUNK)])
        else:
            def step(i, carry):
                e0 = pl.multiple_of(base + i * _GCHUNK, 8)
                pltpu.async_copy(table_hbm.at[idx_v.at[i]], rows_a, sem_a).wait()
                pltpu.sync_copy(rows_a, out_hbm.at[pl.ds(e0, _GCHUNK)])
                return carry

            lax.fori_loop(0, iters, step, 0)

    return gk(table, idx3)


def _sc_scatter_sum(f, idx3, zeros_tile):
    iters = idx3.shape[1]
    per_tile = iters * _SCHUNK
    mesh = plsc.VectorSubcoreMesh(core_axis_name="c", subcore_axis_name="s")

    @functools.partial(
        pl.kernel,
        mesh=mesh,
        out_type=jax.ShapeDtypeStruct((N_NODES, 1024), jnp.float32),
        scratch_types=[
            pltpu.VMEM((iters, _SCHUNK), jnp.int32),
            pltpu.VMEM((_SCHUNK, 128), jnp.float32),
            pltpu.VMEM((_SCHUNK, 128), jnp.float32),
            pltpu.VMEM_SHARED((N_NODES, 128), jnp.float32),
            pltpu.SemaphoreType.DMA,
            pltpu.SemaphoreType.DMA,
        ],
    )
    def sk(f_hbm, idx_hbm, z_hbm, out_hbm, idx_v, val_a, val_b, shared,
           sem_a, sem_b):
        c = lax.axis_index("c")
        s = lax.axis_index("s")
        r0 = pl.multiple_of(s * _NROWS_CP, 8)
        pltpu.sync_copy(idx_hbm.at[s], idx_v)
        for cc in range(4):
            col0 = pl.multiple_of((c * 4 + cc) * 128, 128)

            @pl.when(s < 10)
            def _zero():
                pltpu.sync_copy(z_hbm, shared.at[pl.ds(r0, _NROWS_CP)])

            plsc.subcore_barrier()

            def pair(j, carry):
                i0 = j * 2
                e0 = pl.multiple_of(s * per_tile + i0 * _SCHUNK, 8)
                e1 = pl.multiple_of(s * per_tile + (i0 + 1) * _SCHUNK, 8)
                ca = pltpu.async_copy(
                    f_hbm.at[pl.ds(e0, _SCHUNK), pl.ds(col0, 128)], val_a,
                    sem_a)
                cb = pltpu.async_copy(
                    f_hbm.at[pl.ds(e1, _SCHUNK), pl.ds(col0, 128)], val_b,
                    sem_b)
                ca.wait()
                pltpu.sync_copy(val_a, shared.at[idx_v.at[i0]], add=True)
                cb.wait()
                pltpu.sync_copy(val_b, shared.at[idx_v.at[i0 + 1]], add=True)
                return carry

            lax.fori_loop(0, iters // 2, pair, 0)
            if iters % 2:
                et = pl.multiple_of(s * per_tile + (iters - 1) * _SCHUNK, 8)
                pltpu.async_copy(
                    f_hbm.at[pl.ds(et, _SCHUNK), pl.ds(col0, 128)], val_a,
                    sem_a).wait()
                pltpu.sync_copy(val_a, shared.at[idx_v.at[iters - 1]],
                                add=True)
            plsc.subcore_barrier()

            @pl.when(s < 10)
            def _copy_out():
                pltpu.sync_copy(
                    shared.at[pl.ds(r0, _NROWS_CP)],
                    out_hbm.at[pl.ds(r0, _NROWS_CP), pl.ds(col0, 128)])

            plsc.subcore_barrier()

    return sk(f, idx3, zeros_tile)


def kernel(x, edge_index, edge_attr, params):
    p = params
    src = edge_index[0]
    dst = edge_index[1]

    h_e0 = _run_mlp(edge_attr, p["enc_edge"]["W"], p["enc_edge"]["b"], 640)
    h_n = _run_mlp(x, p["enc_node"]["W"], p["enc_node"]["b"], 1000)
    init_e = h_e0

    w1 = p["edge_model"]["W"][0]
    wa, wb = w1[0:1024], w1[1024:2048]
    c1, c2 = w1[2048:2304], w1[2304:2560]
    ew = {
        "C1": c1, "C2": c2, "Ccomb": c1 + c2,
        "b1": p["edge_model"]["b"][0].reshape(1, -1),
        "W2": p["edge_model"]["W"][1], "b2": p["edge_model"]["b"][1].reshape(1, -1),
        "W3": p["edge_model"]["W"][2], "b3": p["edge_model"]["b"][2].reshape(1, -1),
        "W4": p["edge_model"]["W"][3], "b4": p["edge_model"]["b"][3].reshape(1, -1),
    }
    wfo1, wfi1 = p["flow_out"]["W"][0], p["flow_in"]["W"][0]
    fw = {
        "Do": wfo1[1024:], "bo1": p["flow_out"]["b"][0].reshape(1, -1),
        "Wo2": p["flow_out"]["W"][1], "bo2": p["flow_out"]["b"][1].reshape(1, -1),
        "Di": wfi1[1024:], "bi1": p["flow_in"]["b"][0].reshape(1, -1),
        "Wi2": p["flow_in"]["W"][1], "bi2": p["flow_in"]["b"][1].reshape(1, -1),
    }
    wcat1 = jnp.concatenate([wb, wfo1[:1024], wfi1[:1024]], axis=1)

    src2 = src.reshape(N_EDGES, 1)
    dst2 = dst.reshape(N_EDGES, 1)
    src_g = src.reshape(_NW, N_EDGES // _NW // _GCHUNK, _GCHUNK)
    dst_g = dst.reshape(_NW, N_EDGES // _NW // _GCHUNK, _GCHUNK)
    src_s = src.reshape(16, N_EDGES // 16 // _SCHUNK, _SCHUNK)
    zeros_tile = jnp.zeros((_NROWS_CP, 128), jnp.float32)

    u, td = _node_proj(h_n, wa, wcat1)
    gs = _sc_gather(_pack_cols(u), src_g, 512)
    gd = _sc_gather(_pack_cols(td), dst_g, 1536)
    h_e, f = _edge_step1(gs, gd, init_e, src2, dst2, ew, fw)
    flow = _sc_scatter_sum(f, src_s, zeros_tile)
    h_n = _node_update(flow, p["node_mlp"]["W"][0], p["node_mlp"]["b"][0])

    u2, v2 = _node_proj(h_n, wa, wb)
    gs2 = _sc_gather(_pack_cols(u2), src_g, 512)
    gd2 = _sc_gather(_pack_cols(v2), dst_g, 512)
    out = _edge_step2(gs2, gd2, init_e, h_e, ew, p["classifier"])
    return out
